# baseline (speedup 1.0000x reference)
import functools

import jax
import jax.numpy as jnp
from jax import lax
from jax.experimental import pallas as pl
from jax.experimental.pallas import tpu as pltpu

NUM_IMAGES = 4
H = 540
W = 960
P = 500000


_EW_ROWS = 648


def _ew_body(a_ref, b_ref, c_ref, d_ref, o_ref):
    o_ref[...] = a_ref[...] * b_ref[...] + c_ref[...] * d_ref[...]


def _reweight(res, w, res_s, w_s):
    flat = (NUM_IMAGES * 3 * H, W)
    a = res.reshape(flat)
    b = w.reshape(flat)
    c = res_s.reshape(flat)
    d = w_s.reshape(flat)
    grid = (flat[0] // _EW_ROWS,)
    spec = pl.BlockSpec((_EW_ROWS, W), lambda i: (i, 0))
    out = pl.pallas_call(
        _ew_body,
        grid=grid,
        in_specs=[spec, spec, spec, spec],
        out_specs=spec,
        out_shape=jax.ShapeDtypeStruct(flat, jnp.float32),
    )(a, b, c, d)
    return out.reshape(res.shape)


_TH = H // 2
_TW = W // 16


def _tile_body(x_ref, o_ref):
    x = x_ref[0].astype(jnp.bfloat16)

    pr = lax.broadcasted_iota(jnp.int32, (_TH, H), 0)
    pc = lax.broadcasted_iota(jnp.int32, (_TH, H), 1)
    pmat = jnp.where((pc >> 1) == pr, 1.0, 0.0).astype(jnp.bfloat16)
    y = jax.lax.dot_general(pmat, x, (((1,), (0,)), ((), ())),
                            preferred_element_type=jnp.float32)

    gr = lax.broadcasted_iota(jnp.int32, (W, _TW), 0)
    gc = lax.broadcasted_iota(jnp.int32, (W, _TW), 1)
    gmat = jnp.where((gr >> 4) == gc, 1.0, 0.0).astype(jnp.bfloat16)
    tile = jax.lax.dot_general(y.astype(jnp.bfloat16), gmat,
                               (((1,), (0,)), ((), ())),
                               preferred_element_type=jnp.float32)
    t = tile.astype(jnp.int32)

    col = lax.broadcasted_iota(jnp.int32, (_TH, _TW), 1)
    c_in = t
    for s in (1, 2, 4, 8, 16, 32):
        rolled = pltpu.roll(c_in, s, 1)
        c_in = c_in + jnp.where(col >= s, rolled, 0)

    rt = lax.slice(c_in, (0, _TW - 1), (_TH, _TW))
    row = lax.broadcasted_iota(jnp.int32, (_TH, 1), 0)
    rr = rt
    for s in (1, 2, 4, 8, 16, 32, 64, 128, 256):
        rolled = pltpu.roll(rr, s, 0)
        rr = rr + jnp.where(row >= s, rolled, 0)
    rows_before = rr - rt

    o_ref[0] = c_in + rows_before


def _tile_prefix(n_contrib):
    out = pl.pallas_call(
        _tile_body,
        grid=(NUM_IMAGES,),
        in_specs=[pl.BlockSpec((1, H, W), lambda i: (i, 0, 0))],
        out_specs=pl.BlockSpec((1, _TH, _TW), lambda i: (i, 0, 0)),
        out_shape=jax.ShapeDtypeStruct((NUM_IMAGES, _TH, _TW), jnp.int32),
    )(n_contrib)
    incl = out.reshape(NUM_IMAGES, _TH * _TW)
    num_sparse = incl[:, -1]
    excl = jnp.concatenate(
        [jnp.zeros((NUM_IMAGES, 1), jnp.int32), incl[:, :-1]], axis=1)
    return excl, num_sparse


_VC = 128
_VR = 3968
_VP = _VR * _VC
_VB = _VR // _VC


def _vis_body(x_ref, o_ref):
    rr = lax.broadcasted_iota(jnp.int32, (_VC, _VC), 0)
    cc = lax.broadcasted_iota(jnp.int32, (_VC, _VC), 1)
    umat = jnp.where(rr <= cc, 1.0, 0.0).astype(jnp.bfloat16)
    lmat = jnp.where(cc < rr, 1.0, 0.0).astype(jnp.bfloat16)
    ones = jnp.ones((_VC, _VC), jnp.bfloat16)

    def body(b, carry):
        hb = x_ref[0, pl.ds(b * _VC, _VC), :].astype(jnp.bfloat16)
        local = jax.lax.dot_general(hb, umat, (((1,), (0,)), ((), ())),
                                    preferred_element_type=jnp.float32)
        rs = jax.lax.dot_general(hb, ones, (((1,), (0,)), ((), ())),
                                 preferred_element_type=jnp.float32)
        before = jax.lax.dot_general(lmat, rs.astype(jnp.bfloat16),
                                     (((1,), (0,)), ((), ())),
                                     preferred_element_type=jnp.float32)
        out = local + before + carry
        o_ref[0, pl.ds(b * _VC, _VC), :] = out.astype(jnp.int32) - 1
        blocktot = lax.slice(before + rs, (_VC - 1, 0), (_VC, 1))
        return carry + blocktot

    lax.fori_loop(0, _VB, body, jnp.zeros((1, 1), jnp.float32))


def _vis_map(is_hit):
    padded = jnp.pad(is_hit, ((0, 0), (0, _VP - P)))
    x = padded.reshape(NUM_IMAGES, _VR, _VC)
    out = pl.pallas_call(
        _vis_body,
        grid=(NUM_IMAGES,),
        in_specs=[pl.BlockSpec((1, _VR, _VC), lambda i: (i, 0, 0))],
        out_specs=pl.BlockSpec((1, _VR, _VC), lambda i: (i, 0, 0)),
        out_shape=jax.ShapeDtypeStruct((NUM_IMAGES, _VR, _VC), jnp.int32),
    )(x)
    vmap = out.reshape(NUM_IMAGES, _VP)[:, :P]
    num_visible = vmap[:, -1] + 1
    return vmap, num_visible


@jax.jit
def kernel(n_contrib_vol_rend, is_gaussian_hit, residuals, weights,
           residuals_ssim, weights_ssim):
    r = _reweight(residuals, weights, residuals_ssim, weights_ssim)
    n_contrib_prefix_sum, num_sparse_gaussians = _tile_prefix(n_contrib_vol_rend)
    map_visible_gaussians, num_visible_gaussians = _vis_map(is_gaussian_hit)
    return (r, n_contrib_prefix_sum, num_sparse_gaussians,
            map_visible_gaussians, num_visible_gaussians)


# speedup vs baseline: 1.3522x; 1.3522x over previous
import jax
import jax.numpy as jnp
from jax import lax
from jax.experimental import pallas as pl
from jax.experimental.pallas import tpu as pltpu

NUM_IMAGES = 4
H = 540
W = 960
P = 500000


def _ew_body(a_ref, b_ref, c_ref, d_ref, o_ref):
    o_ref[...] = a_ref[...] * b_ref[...] + c_ref[...] * d_ref[...]


def _reweight(res, w, res_s, w_s):
    flat = (NUM_IMAGES * 3, H, W)
    a = res.reshape(flat)
    b = w.reshape(flat)
    c = res_s.reshape(flat)
    d = w_s.reshape(flat)
    spec = pl.BlockSpec((1, H, W), lambda i: (i, 0, 0))
    out = pl.pallas_call(
        _ew_body,
        grid=(flat[0],),
        in_specs=[spec, spec, spec, spec],
        out_specs=spec,
        out_shape=jax.ShapeDtypeStruct(flat, jnp.float32),
    )(a, b, c, d)
    return out.reshape(res.shape)


_TH = H // 2
_TW = W // 16
_NT = _TH * _TW


def _tile_body(x_ref, excl_ref, num_ref):
    x = x_ref[0].astype(jnp.bfloat16)

    pr = lax.broadcasted_iota(jnp.int32, (_TH, H), 0)
    pc = lax.broadcasted_iota(jnp.int32, (_TH, H), 1)
    pmat = jnp.where((pc >> 1) == pr, 1.0, 0.0).astype(jnp.bfloat16)
    y = jax.lax.dot_general(pmat, x, (((1,), (0,)), ((), ())),
                            preferred_element_type=jnp.float32)

    gr = lax.broadcasted_iota(jnp.int32, (W, _TW), 0)
    gc = lax.broadcasted_iota(jnp.int32, (W, _TW), 1)
    gmat = jnp.where((gr >> 4) == gc, 1.0, 0.0).astype(jnp.bfloat16)
    tile = jax.lax.dot_general(y.astype(jnp.bfloat16), gmat,
                               (((1,), (0,)), ((), ())),
                               preferred_element_type=jnp.float32)
    t = tile.astype(jnp.int32)

    col = lax.broadcasted_iota(jnp.int32, (_TH, _TW), 1)
    for s in (1, 2, 4, 8, 16, 32):
        t = t + jnp.where(col >= s, pltpu.roll(t, s, 1), 0)

    rt = lax.slice(t, (0, _TW - 1), (_TH, _TW))
    row = lax.broadcasted_iota(jnp.int32, (_TH, 1), 0)
    rr = rt
    for s in (1, 2, 4, 8, 16, 32, 64, 128, 256):
        rr = rr + jnp.where(row >= s, pltpu.roll(rr, s, 0), 0)
    incl = t + (rr - rt)

    num_ref[0] = lax.slice(incl, (_TH - 1, _TW - 1), (_TH, _TW))

    prev_row_incl = jnp.where(row >= 1, pltpu.roll(rr, 1, 0), 0)
    excl = jnp.where(col >= 1, pltpu.roll(incl, 1, 1), prev_row_incl)
    excl_ref[0] = excl


def _tile_prefix(n_contrib):
    excl, num = pl.pallas_call(
        _tile_body,
        grid=(NUM_IMAGES,),
        in_specs=[pl.BlockSpec((1, H, W), lambda i: (i, 0, 0))],
        out_specs=[pl.BlockSpec((1, _TH, _TW), lambda i: (i, 0, 0)),
                   pl.BlockSpec((1, 1, 1), lambda i: (i, 0, 0))],
        out_shape=[jax.ShapeDtypeStruct((NUM_IMAGES, _TH, _TW), jnp.int32),
                   jax.ShapeDtypeStruct((NUM_IMAGES, 1, 1), jnp.int32)],
    )(n_contrib)
    return excl.reshape(NUM_IMAGES, _NT), num.reshape(NUM_IMAGES)


_CH = 16384
_NCH = -(-P // _CH)
_LAST_COL = (P - 1) - (_NCH - 1) * _CH


def _vis_body(x_ref, o_ref, num_ref, carry):
    j = pl.program_id(0)

    @pl.when(j == 0)
    def _init():
        carry[...] = jnp.zeros_like(carry)

    x = x_ref[...]
    col = lax.broadcasted_iota(jnp.int32, (NUM_IMAGES, _CH), 1)
    for s in (1, 2, 4, 8, 16, 32, 64, 128, 256, 512, 1024, 2048, 4096, 8192):
        x = x + jnp.where(col >= s, pltpu.roll(x, s, 1), 0)
    x = x + carry[0:NUM_IMAGES, 0:1]
    o_ref[...] = x - 1
    carry[0:NUM_IMAGES, 0:1] = lax.slice(x, (0, _CH - 1), (NUM_IMAGES, _CH))

    @pl.when(j == _NCH - 1)
    def _num():
        num_ref[...] = lax.slice(
            x, (0, _LAST_COL), (NUM_IMAGES, _LAST_COL + 1)).reshape(
                NUM_IMAGES, 1, 1)


def _vis_map(is_hit):
    vmap, num = pl.pallas_call(
        _vis_body,
        grid=(_NCH,),
        in_specs=[pl.BlockSpec((NUM_IMAGES, _CH), lambda j: (0, j))],
        out_specs=[pl.BlockSpec((NUM_IMAGES, _CH), lambda j: (0, j)),
                   pl.BlockSpec((NUM_IMAGES, 1, 1), lambda j: (0, 0, 0))],
        out_shape=[jax.ShapeDtypeStruct((NUM_IMAGES, P), jnp.int32),
                   jax.ShapeDtypeStruct((NUM_IMAGES, 1, 1), jnp.int32)],
        scratch_shapes=[pltpu.VMEM((8, 128), jnp.int32)],
    )(is_hit)
    return vmap, num.reshape(NUM_IMAGES)


@jax.jit
def kernel(n_contrib_vol_rend, is_gaussian_hit, residuals, weights,
           residuals_ssim, weights_ssim):
    r = _reweight(residuals, weights, residuals_ssim, weights_ssim)
    n_contrib_prefix_sum, num_sparse_gaussians = _tile_prefix(n_contrib_vol_rend)
    map_visible_gaussians, num_visible_gaussians = _vis_map(is_gaussian_hit)
    return (r, n_contrib_prefix_sum, num_sparse_gaussians,
            map_visible_gaussians, num_visible_gaussians)


# speedup vs baseline: 2.8778x; 2.1282x over previous
import jax
import jax.numpy as jnp
from jax import lax
from jax.experimental import pallas as pl
from jax.experimental.pallas import tpu as pltpu

NUM_IMAGES = 4
H = 540
W = 960
P = 500000


_EW_BH = 270


def _ew_body(a_ref, b_ref, c_ref, d_ref, o_ref):
    o_ref[...] = a_ref[...] * b_ref[...] + c_ref[...] * d_ref[...]


def _t_view(x):
    return jnp.transpose(x, (1, 2, 0, 3)).reshape(3 * H, NUM_IMAGES, W)


def _reweight(res, w, res_s, w_s):
    a, b, c, d = _t_view(res), _t_view(w), _t_view(res_s), _t_view(w_s)
    spec = pl.BlockSpec((_EW_BH, NUM_IMAGES, W), lambda i: (i, 0, 0))
    out = pl.pallas_call(
        _ew_body,
        grid=(3 * H // _EW_BH,),
        in_specs=[spec, spec, spec, spec],
        out_specs=spec,
        out_shape=jax.ShapeDtypeStruct((3 * H, NUM_IMAGES, W), jnp.float32),
    )(a, b, c, d)
    return jnp.transpose(out.reshape(3, H, NUM_IMAGES, W), (2, 0, 1, 3))


_TH = H // 2
_TW = W // 16
_NT = _TH * _TW


def _tile_body(x_ref, excl_ref, num_ref):
    pr = lax.broadcasted_iota(jnp.int32, (_TH, H), 0)
    pc = lax.broadcasted_iota(jnp.int32, (_TH, H), 1)
    pmat = jnp.where((pc >> 1) == pr, 1.0, 0.0).astype(jnp.bfloat16)
    gr = lax.broadcasted_iota(jnp.int32, (W, _TW), 0)
    gc = lax.broadcasted_iota(jnp.int32, (W, _TW), 1)
    gmat = jnp.where((gr >> 4) == gc, 1.0, 0.0).astype(jnp.bfloat16)

    col = lax.broadcasted_iota(jnp.int32, (_TH, _TW), 1)
    row = lax.broadcasted_iota(jnp.int32, (_TH, 1), 0)

    for i in range(NUM_IMAGES):
        x = x_ref[:, i, :].astype(jnp.bfloat16)
        y = jax.lax.dot_general(pmat, x, (((1,), (0,)), ((), ())),
                                preferred_element_type=jnp.float32)
        tile = jax.lax.dot_general(y.astype(jnp.bfloat16), gmat,
                                   (((1,), (0,)), ((), ())),
                                   preferred_element_type=jnp.float32)
        t = tile.astype(jnp.int32)

        for s in (1, 2, 4, 8, 16, 32):
            t = t + jnp.where(col >= s, pltpu.roll(t, s, 1), 0)

        rt = lax.slice(t, (0, _TW - 1), (_TH, _TW))
        rr = rt
        for s in (1, 2, 4, 8, 16, 32, 64, 128, 256):
            rr = rr + jnp.where(row >= s, pltpu.roll(rr, s, 0), 0)
        incl = t + (rr - rt)

        num_ref[i] = lax.slice(incl, (_TH - 1, _TW - 1), (_TH, _TW))

        prev_row_incl = jnp.where(row >= 1, pltpu.roll(rr, 1, 0), 0)
        excl_ref[i] = jnp.where(col >= 1, pltpu.roll(incl, 1, 1), prev_row_incl)


def _tile_prefix(n_contrib):
    nt = jnp.transpose(n_contrib, (1, 0, 2))
    excl, num = pl.pallas_call(
        _tile_body,
        grid=(1,),
        in_specs=[pl.BlockSpec((H, NUM_IMAGES, W), lambda i: (0, 0, 0))],
        out_specs=[pl.BlockSpec((NUM_IMAGES, _TH, _TW), lambda i: (0, 0, 0)),
                   pl.BlockSpec((NUM_IMAGES, 1, 1), lambda i: (0, 0, 0))],
        out_shape=[jax.ShapeDtypeStruct((NUM_IMAGES, _TH, _TW), jnp.int32),
                   jax.ShapeDtypeStruct((NUM_IMAGES, 1, 1), jnp.int32)],
    )(nt)
    return excl.reshape(NUM_IMAGES, _NT), num.reshape(NUM_IMAGES)


_CH = 16384
_NCH = -(-P // _CH)
_LAST_COL = (P - 1) - (_NCH - 1) * _CH


def _vis_body(x_ref, o_ref, num_ref, carry):
    j = pl.program_id(0)

    @pl.when(j == 0)
    def _init():
        carry[...] = jnp.zeros_like(carry)

    x = x_ref[...]
    col = lax.broadcasted_iota(jnp.int32, (NUM_IMAGES, _CH), 1)
    for s in (1, 2, 4, 8, 16, 32, 64, 128, 256, 512, 1024, 2048, 4096, 8192):
        x = x + jnp.where(col >= s, pltpu.roll(x, s, 1), 0)
    x = x + carry[0:NUM_IMAGES, 0:1]
    o_ref[...] = x - 1
    carry[0:NUM_IMAGES, 0:1] = lax.slice(x, (0, _CH - 1), (NUM_IMAGES, _CH))

    @pl.when(j == _NCH - 1)
    def _num():
        num_ref[...] = lax.slice(
            x, (0, _LAST_COL), (NUM_IMAGES, _LAST_COL + 1)).reshape(
                NUM_IMAGES, 1, 1)


def _vis_map(is_hit):
    vmap, num = pl.pallas_call(
        _vis_body,
        grid=(_NCH,),
        in_specs=[pl.BlockSpec((NUM_IMAGES, _CH), lambda j: (0, j))],
        out_specs=[pl.BlockSpec((NUM_IMAGES, _CH), lambda j: (0, j)),
                   pl.BlockSpec((NUM_IMAGES, 1, 1), lambda j: (0, 0, 0))],
        out_shape=[jax.ShapeDtypeStruct((NUM_IMAGES, P), jnp.int32),
                   jax.ShapeDtypeStruct((NUM_IMAGES, 1, 1), jnp.int32)],
        scratch_shapes=[pltpu.VMEM((8, 128), jnp.int32)],
    )(is_hit)
    return vmap, num.reshape(NUM_IMAGES)


@jax.jit
def kernel(n_contrib_vol_rend, is_gaussian_hit, residuals, weights,
           residuals_ssim, weights_ssim):
    r = _reweight(residuals, weights, residuals_ssim, weights_ssim)
    n_contrib_prefix_sum, num_sparse_gaussians = _tile_prefix(n_contrib_vol_rend)
    map_visible_gaussians, num_visible_gaussians = _vis_map(is_gaussian_hit)
    return (r, n_contrib_prefix_sum, num_sparse_gaussians,
            map_visible_gaussians, num_visible_gaussians)


# speedup vs baseline: 3.1819x; 1.1057x over previous
import jax
import jax.numpy as jnp
from jax import lax
from jax.experimental import pallas as pl
from jax.experimental.pallas import tpu as pltpu

NUM_IMAGES = 4
H = 540
W = 960
P = 500000


_EW_BH = 270


def _ew_body(a_ref, b_ref, c_ref, d_ref, o_ref):
    o_ref[...] = a_ref[...] * b_ref[...] + c_ref[...] * d_ref[...]


def _t_view(x):
    return jnp.transpose(x, (1, 2, 0, 3)).reshape(3 * H, NUM_IMAGES, W)


def _reweight(res, w, res_s, w_s):
    a, b, c, d = _t_view(res), _t_view(w), _t_view(res_s), _t_view(w_s)
    spec = pl.BlockSpec((_EW_BH, NUM_IMAGES, W), lambda i: (i, 0, 0))
    out = pl.pallas_call(
        _ew_body,
        grid=(3 * H // _EW_BH,),
        in_specs=[spec, spec, spec, spec],
        out_specs=spec,
        out_shape=jax.ShapeDtypeStruct((3 * H, NUM_IMAGES, W), jnp.float32),
    )(a, b, c, d)
    return jnp.transpose(out.reshape(3, H, NUM_IMAGES, W), (2, 0, 1, 3))


_TH = H // 2
_TW = W // 16
_NT = _TH * _TW


_TR = _TH * NUM_IMAGES


def _tile_body(x_ref, excl_ref, num_ref):
    a = x_ref[:, 0].astype(jnp.bfloat16)
    b = x_ref[:, 1].astype(jnp.bfloat16)
    p = (a + b).reshape(_TR, W)

    gr = lax.broadcasted_iota(jnp.int32, (W, _TW), 0)
    gc = lax.broadcasted_iota(jnp.int32, (W, _TW), 1)
    gmat = jnp.where((gr >> 4) == gc, 1.0, 0.0).astype(jnp.bfloat16)
    t = jax.lax.dot_general(p, gmat, (((1,), (0,)), ((), ())),
                            preferred_element_type=jnp.float32)
    t = t.astype(jnp.int32)

    col = lax.broadcasted_iota(jnp.int32, (_TR, _TW), 1)
    for s in (1, 2, 4, 8, 16, 32):
        t = t + jnp.where(col >= s, pltpu.roll(t, s, 1), 0)

    rt = lax.slice(t, (0, _TW - 1), (_TR, _TW))
    row = lax.broadcasted_iota(jnp.int32, (_TR, 1), 0)
    rr = rt
    for s in (4, 8, 16, 32, 64, 128, 256, 512, 1024):
        rr = rr + jnp.where(row >= s, pltpu.roll(rr, s, 0), 0)
    incl = t + (rr - rt)

    num_ref[...] = lax.slice(incl, (_TR - NUM_IMAGES, _TW - 1),
                             (_TR, _TW))

    prev_row_incl = jnp.where(row >= 4, pltpu.roll(rr, 4, 0), 0)
    excl_ref[...] = jnp.where(col >= 1, pltpu.roll(incl, 1, 1), prev_row_incl)


def _tile_prefix(n_contrib):
    nt = jnp.transpose(n_contrib, (1, 0, 2)).reshape(_TH, 2, NUM_IMAGES, W)
    excl, num = pl.pallas_call(
        _tile_body,
        grid=(1,),
        in_specs=[pl.BlockSpec((_TH, 2, NUM_IMAGES, W), lambda i: (0, 0, 0, 0))],
        out_specs=[pl.BlockSpec((_TR, _TW), lambda i: (0, 0)),
                   pl.BlockSpec((NUM_IMAGES, 1), lambda i: (0, 0))],
        out_shape=[jax.ShapeDtypeStruct((_TR, _TW), jnp.int32),
                   jax.ShapeDtypeStruct((NUM_IMAGES, 1), jnp.int32)],
    )(nt)
    excl_flat = jnp.transpose(excl.reshape(_TH, NUM_IMAGES, _TW),
                              (1, 0, 2)).reshape(NUM_IMAGES, _NT)
    return excl_flat, num.reshape(NUM_IMAGES)


_CH = 16384
_NCH = -(-P // _CH)
_LAST_COL = (P - 1) - (_NCH - 1) * _CH


def _vis_body(x_ref, o_ref, num_ref, carry):
    j = pl.program_id(0)

    @pl.when(j == 0)
    def _init():
        carry[...] = jnp.zeros_like(carry)

    x = x_ref[...]
    col = lax.broadcasted_iota(jnp.int32, (NUM_IMAGES, _CH), 1)
    for s in (1, 2, 4, 8, 16, 32, 64, 128, 256, 512, 1024, 2048, 4096, 8192):
        x = x + jnp.where(col >= s, pltpu.roll(x, s, 1), 0)
    x = x + carry[0:NUM_IMAGES, 0:1]
    o_ref[...] = x - 1
    carry[0:NUM_IMAGES, 0:1] = lax.slice(x, (0, _CH - 1), (NUM_IMAGES, _CH))

    @pl.when(j == _NCH - 1)
    def _num():
        num_ref[...] = lax.slice(
            x, (0, _LAST_COL), (NUM_IMAGES, _LAST_COL + 1)).reshape(
                NUM_IMAGES, 1, 1)


def _vis_map(is_hit):
    vmap, num = pl.pallas_call(
        _vis_body,
        grid=(_NCH,),
        in_specs=[pl.BlockSpec((NUM_IMAGES, _CH), lambda j: (0, j))],
        out_specs=[pl.BlockSpec((NUM_IMAGES, _CH), lambda j: (0, j)),
                   pl.BlockSpec((NUM_IMAGES, 1, 1), lambda j: (0, 0, 0))],
        out_shape=[jax.ShapeDtypeStruct((NUM_IMAGES, P), jnp.int32),
                   jax.ShapeDtypeStruct((NUM_IMAGES, 1, 1), jnp.int32)],
        scratch_shapes=[pltpu.VMEM((8, 128), jnp.int32)],
    )(is_hit)
    return vmap, num.reshape(NUM_IMAGES)


@jax.jit
def kernel(n_contrib_vol_rend, is_gaussian_hit, residuals, weights,
           residuals_ssim, weights_ssim):
    r = _reweight(residuals, weights, residuals_ssim, weights_ssim)
    n_contrib_prefix_sum, num_sparse_gaussians = _tile_prefix(n_contrib_vol_rend)
    map_visible_gaussians, num_visible_gaussians = _vis_map(is_gaussian_hit)
    return (r, n_contrib_prefix_sum, num_sparse_gaussians,
            map_visible_gaussians, num_visible_gaussians)
